# Initial kernel scaffold; baseline (speedup 1.0000x reference)
import jax
import jax.numpy as jnp
from jax.experimental import pallas as pl


def kernel(input, adj, W1, b1, W2, b2):
    raise NotImplementedError("write your pallas kernel here")


# speedup vs baseline: 2.3789x; 2.3789x over previous
import jax
import jax.numpy as jnp
from jax.experimental import pallas as pl
from jax.experimental.pallas import tpu as pltpu

_R = 200


def _p1_kernel(a_ref, x_ref, w1_ref, s_ref, z1_ref):
    a = a_ref[...]
    deg = jnp.sum(a, axis=1, keepdims=True) + 1.0
    s = jax.lax.rsqrt(jnp.maximum(deg, 1e-12))
    s_ref[...] = s
    y = jnp.dot(x_ref[...], w1_ref[...], preferred_element_type=jnp.float32)
    z1_ref[...] = y * s


def _p2_kernel(a_ref, z1_ref, s_ref, b1_ref, w2_ref, z2_ref):
    i = pl.program_id(0)
    acc = jnp.dot(a_ref[...], z1_ref[...], preferred_element_type=jnp.float32)
    acc = acc + z1_ref[pl.ds(i * _R, _R), :]
    s = s_ref[...]
    h = jnp.maximum(acc * s + b1_ref[...], 0.0)
    z2 = jnp.dot(h, w2_ref[...], preferred_element_type=jnp.float32)
    z2_ref[...] = z2 * s


def _p3_kernel(a_ref, z2_ref, s_ref, b2_ref, o_ref):
    i = pl.program_id(0)
    acc = jnp.dot(a_ref[...], z2_ref[...], preferred_element_type=jnp.float32)
    acc = acc + z2_ref[pl.ds(i * _R, _R), :]
    logits = acc * s_ref[...] + b2_ref[...]
    m = jnp.max(logits, axis=-1, keepdims=True)
    e = jnp.exp(logits - m)
    o_ref[...] = e / jnp.sum(e, axis=-1, keepdims=True)


def kernel(input, adj, W1, b1, W2, b2):
    n = adj.shape[0]
    d_hidden = W1.shape[1]
    d_out = W2.shape[1]
    grid = (n // _R,)
    row_block = lambda r: (r, 0)
    full = lambda r: (0, 0)

    s, z1 = pl.pallas_call(
        _p1_kernel,
        grid=grid,
        in_specs=[
            pl.BlockSpec((_R, n), row_block),
            pl.BlockSpec((_R, n), row_block),
            pl.BlockSpec((n, d_hidden), full),
        ],
        out_specs=[
            pl.BlockSpec((_R, 1), row_block),
            pl.BlockSpec((_R, d_hidden), row_block),
        ],
        out_shape=[
            jax.ShapeDtypeStruct((n, 1), jnp.float32),
            jax.ShapeDtypeStruct((n, d_hidden), jnp.float32),
        ],
        compiler_params=pltpu.CompilerParams(
            dimension_semantics=("arbitrary",),
        ),
    )(adj, input, W1)

    z2 = pl.pallas_call(
        _p2_kernel,
        grid=grid,
        in_specs=[
            pl.BlockSpec((_R, n), row_block),
            pl.BlockSpec((n, d_hidden), full),
            pl.BlockSpec((_R, 1), row_block),
            pl.BlockSpec((1, d_hidden), full),
            pl.BlockSpec((d_hidden, d_out), full),
        ],
        out_specs=pl.BlockSpec((_R, d_out), row_block),
        out_shape=jax.ShapeDtypeStruct((n, d_out), jnp.float32),
        compiler_params=pltpu.CompilerParams(
            dimension_semantics=("arbitrary",),
        ),
    )(adj, z1, s, b1.reshape(1, d_hidden), W2)

    out = pl.pallas_call(
        _p3_kernel,
        grid=grid,
        in_specs=[
            pl.BlockSpec((_R, n), row_block),
            pl.BlockSpec((n, d_out), full),
            pl.BlockSpec((_R, 1), row_block),
            pl.BlockSpec((1, d_out), full),
        ],
        out_specs=pl.BlockSpec((_R, d_out), row_block),
        out_shape=jax.ShapeDtypeStruct((n, d_out), jnp.float32),
        compiler_params=pltpu.CompilerParams(
            dimension_semantics=("arbitrary",),
        ),
    )(adj, z2, s, b2.reshape(1, d_out))

    return out
